# baseline (speedup 1.0000x reference)
import functools

import jax
import jax.numpy as jnp
from jax import lax
from jax.experimental import pallas as pl
from jax.experimental.pallas import tpu as pltpu
from jax.experimental.pallas import tpu_sc as plsc

S = 2048
M = 768
E = 8
MID = 3072
OUT = 768
C = 512
EC = E * C
EP = 16
LANES = 128
NW = 32
L = 16
ROWS_DISP = EC // NW
ROWS_COMB = S // NW
BM = 512
KM = MID // BM
SENT_TOK = S
SENT_SLOT = EC


def _gating_body(x_ref, wg_ref, dst_ref, gate_ref, laux_ref):
    x = x_ref[...]
    wg = wg_ref[...]
    logits = jnp.dot(x, wg, preferred_element_type=jnp.float32)
    col = lax.broadcasted_iota(jnp.int32, (S, LANES), 1)
    valid = col < E
    logits = jnp.where(valid, logits, -1e30)
    mx = jnp.max(logits, axis=1, keepdims=True)
    p = jnp.exp(logits - mx)
    probs = p / jnp.sum(p, axis=1, keepdims=True)
    pm = jnp.max(probs, axis=1, keepdims=True)
    is_max = probs == pm
    eidx = jnp.min(jnp.where(is_max, col, LANES), axis=1, keepdims=True)
    onehot = jnp.where((col == eidx) & valid, 1.0, 0.0)
    c = onehot
    k = 1
    while k < S:
        shifted = jnp.concatenate(
            [jnp.zeros((k, LANES), jnp.float32), c[: S - k, :]], axis=0)
        c = c + shifted
        k *= 2
    pos = jnp.sum(c * onehot, axis=1, keepdims=True).astype(jnp.int32) - 1
    kept = pos < C
    dst = jnp.where(kept, eidx * C + pos, SENT_SLOT)
    dst_ref[...] = dst.astype(jnp.int32).reshape(S // LANES, LANES)
    gate = jnp.sum(probs * onehot, axis=1, keepdims=True)
    gate_ref[...] = gate.reshape(S // LANES, LANES)
    counts = jnp.sum(onehot, axis=0, keepdims=True)
    me = jnp.sum(probs, axis=0, keepdims=True) / S
    laux_ref[...] = jnp.sum(me * (counts / S), axis=1, keepdims=True) * E


def _gating(x, wg_pad):
    return pl.pallas_call(
        _gating_body,
        out_shape=(
            jax.ShapeDtypeStruct((S // LANES, LANES), jnp.int32),
            jax.ShapeDtypeStruct((S // LANES, LANES), jnp.float32),
            jax.ShapeDtypeStruct((1, 1), jnp.float32),
        ),
    )(x, wg_pad)


RD2 = (EC // 2) // NW


def _make_dispatch_body(half):
    def _dispatch_body(dst_hbm, gate_hbm, feat_hbm, disp_hbm, scale_hbm,
                       dst_v, gate_v, slot_v, scale_v, rows_v, sem):
        wid = lax.axis_index("s") * 2 + lax.axis_index("c")
        lbase = wid * RD2
        gbase = half * (EC // 2) + lbase
        pltpu.sync_copy(dst_hbm, dst_v)
        pltpu.sync_copy(gate_hbm, gate_v)
        for j in range(RD2 // L):
            dflt = (lax.iota(jnp.int32, L) + (gbase + j * L)) & (S - 1)
            slot_v[pl.ds(j * L, L)] = dflt

        def build(i, _):
            idx = dst_v[pl.ds(i * L, L)] - gbase
            m = (idx >= 0) & (idx < RD2)
            tok = lax.iota(jnp.int32, L) + i * L
            plsc.store_scatter(slot_v, [idx], tok, mask=m)
            return 0
        lax.fori_loop(0, S // L, build, 0)

        for j in range(RD2 // L):
            sidx = slot_v[pl.ds(j * L, L)]
            scale_v[pl.ds(j * L, L)] = plsc.load_gather(gate_v, [sidx])
        pltpu.sync_copy(scale_v, scale_hbm.at[pl.ds(lbase, RD2)])

        pltpu.async_copy(feat_hbm.at[slot_v], rows_v, sem).wait()
        pltpu.sync_copy(rows_v, disp_hbm.at[pl.ds(lbase, RD2)])
    return _dispatch_body


def _dispatch_half(half, dst, gate, feat):
    mesh = plsc.VectorSubcoreMesh(core_axis_name="c", subcore_axis_name="s")
    return pl.kernel(
        _make_dispatch_body(half),
        out_type=(
            jax.ShapeDtypeStruct((EC // 2, M), jnp.float32),
            jax.ShapeDtypeStruct((EC // 2,), jnp.float32),
        ),
        mesh=mesh,
        compiler_params=pltpu.CompilerParams(needs_layout_passes=False),
        scratch_types=[
            pltpu.VMEM((S,), jnp.int32),
            pltpu.VMEM((S,), jnp.float32),
            pltpu.VMEM((RD2,), jnp.int32),
            pltpu.VMEM((RD2,), jnp.float32),
            pltpu.VMEM((RD2, M), jnp.float32),
            pltpu.SemaphoreType.DMA,
        ],
    )(dst, gate, feat)


def _ffn_body(x_ref, w1_ref, b1_ref, w2_ref, b2_ref, sc_ref, o_ref, acc_ref):
    k = pl.program_id(1)

    @pl.when(k == 0)
    def _():
        acc_ref[...] = jnp.zeros_like(acc_ref)

    h = jnp.dot(x_ref[...].astype(jnp.bfloat16), w1_ref[0].astype(jnp.bfloat16),
                preferred_element_type=jnp.float32)
    h = jnp.maximum(h + b1_ref[0, 0, 0][None, :], 0.0)
    acc_ref[...] += jnp.dot(h.astype(jnp.bfloat16), w2_ref[0].astype(jnp.bfloat16),
                            preferred_element_type=jnp.float32)

    @pl.when(k == KM - 1)
    def _():
        o_ref[...] = (acc_ref[...] + b2_ref[0, 0][None, :]) * sc_ref[...]


def _ffn_body_alias(x_ref, w1_ref, b1_ref, w2_ref, b2_ref, sc_ref, prev_ref,
                    o_ref, acc_ref):
    del prev_ref
    _ffn_body(x_ref, w1_ref, b1_ref, w2_ref, b2_ref, sc_ref, o_ref, acc_ref)


def _ffn_half(disp, scale, w1, b1, w2, b2, e0, nb, prev=None):
    nx = disp.shape[0] // C
    cx = lambda e: jnp.minimum(e, nx - 1)
    cw = lambda e: jnp.minimum(e0 + e, E - 1)
    in_specs = [
        pl.BlockSpec((C, M), lambda e, k: (cx(e), 0)),
        pl.BlockSpec((1, M, BM), lambda e, k: (cw(e), 0, k)),
        pl.BlockSpec((1, 1, 1, BM), lambda e, k: (cw(e), k, 0, 0)),
        pl.BlockSpec((1, BM, OUT), lambda e, k: (cw(e), k, 0)),
        pl.BlockSpec((1, 1, OUT), lambda e, k: (cw(e), 0, 0)),
        pl.BlockSpec((C, 1), lambda e, k: (e, 0)),
    ]
    args = [disp, w1, b1.reshape(E, KM, 1, BM), w2, b2.reshape(E, 1, OUT),
            scale]
    body = _ffn_body
    aliases = {}
    if prev is not None:
        in_specs.append(pl.BlockSpec(memory_space=pl.ANY))
        args.append(prev)
        body = _ffn_body_alias
        aliases = {6: 0}
    return pl.pallas_call(
        body,
        grid=(nb, KM),
        in_specs=in_specs,
        out_specs=pl.BlockSpec((C, OUT), lambda e, k: (e0 + e, 0)),
        out_shape=jax.ShapeDtypeStruct((EC + C, OUT), jnp.float32),
        scratch_shapes=[pltpu.VMEM((C, OUT), jnp.float32)],
        input_output_aliases=aliases,
    )(*args)


def _combine_body(dst_hbm, sout_hbm, out_hbm, idx_v, rows_v, sem):
    wid = lax.axis_index("s") * 2 + lax.axis_index("c")
    base = wid * ROWS_COMB
    pltpu.sync_copy(dst_hbm.at[pl.ds(base, ROWS_COMB)], idx_v)
    pltpu.async_copy(sout_hbm.at[idx_v], rows_v, sem).wait()
    pltpu.sync_copy(rows_v, out_hbm.at[pl.ds(base, ROWS_COMB)])


def _combine(dst, sout):
    mesh = plsc.VectorSubcoreMesh(core_axis_name="c", subcore_axis_name="s")
    return pl.kernel(
        _combine_body,
        out_type=jax.ShapeDtypeStruct((S, OUT), jnp.float32),
        mesh=mesh,
        compiler_params=pltpu.CompilerParams(needs_layout_passes=False),
        scratch_types=[
            pltpu.VMEM((ROWS_COMB,), jnp.int32),
            pltpu.VMEM((ROWS_COMB, OUT), jnp.float32),
            pltpu.SemaphoreType.DMA,
        ],
    )(dst, sout)


@jax.jit
def kernel(hidden_states, Wg, W1, b1, W2, b2):
    b, t, m = hidden_states.shape
    feat = hidden_states.reshape(S, M)
    wg_pad = jnp.zeros((M, LANES), jnp.float32).at[:, :E].set(Wg)
    dst2, gate2, laux = _gating(feat, wg_pad)
    dst = dst2.reshape(S)
    gate = gate2.reshape(S)
    dispA, scaleA = _dispatch_half(0, dst, gate, feat)
    dispB, scaleB = _dispatch_half(1, dst, gate, feat)
    scaleB_full = jnp.concatenate([scaleB, jnp.zeros((C,), jnp.float32)])
    soutA = _ffn_half(dispA, scaleA.reshape(EC // 2, 1),
                      W1, b1, W2, b2, e0=0, nb=E // 2)
    sout = _ffn_half(dispB, scaleB_full.reshape(EC // 2 + C, 1),
                     W1, b1, W2, b2, e0=E // 2, nb=E // 2 + 1, prev=soutA)
    combined = _combine(dst, sout)
    return combined.reshape(b, t, OUT), laux.reshape(())


# speedup vs baseline: 1.0704x; 1.0704x over previous
import jax
import jax.numpy as jnp
from jax import lax
from jax.experimental import pallas as pl
from jax.experimental.pallas import tpu as pltpu
from jax.experimental.pallas import tpu_sc as plsc

S = 2048
M = 768
E = 8
MID = 3072
OUT = 768
C = 512
EC = E * C
LANES = 128
NW = 32
L = 16
ROWS_DISP = EC // NW
ROWS_COMB = S // NW
BM = 512
KM = MID // BM
SENT_SLOT = EC


def _gating_body(x_ref, wg_ref, dst_ref, gate_ref, laux_ref):
    x = x_ref[...]
    wg = wg_ref[...]
    logits = jnp.dot(x, wg, preferred_element_type=jnp.float32)
    col = lax.broadcasted_iota(jnp.int32, (S, LANES), 1)
    valid = col < E
    logits = jnp.where(valid, logits, -1e30)
    mx = jnp.max(logits, axis=1, keepdims=True)
    p = jnp.exp(logits - mx)
    probs = p / jnp.sum(p, axis=1, keepdims=True)
    pm = jnp.max(probs, axis=1, keepdims=True)
    is_max = probs == pm
    eidx = jnp.min(jnp.where(is_max, col, LANES), axis=1, keepdims=True)
    onehot = jnp.where((col == eidx) & valid, 1.0, 0.0)
    c = onehot
    k = 1
    while k < S:
        shifted = jnp.concatenate(
            [jnp.zeros((k, LANES), jnp.float32), c[: S - k, :]], axis=0)
        c = c + shifted
        k *= 2
    pos = jnp.sum(c * onehot, axis=1, keepdims=True).astype(jnp.int32) - 1
    kept = pos < C
    dst = jnp.where(kept, eidx * C + pos, SENT_SLOT)
    dst_ref[...] = dst.astype(jnp.int32).reshape(S // LANES, LANES)
    gate = jnp.sum(probs * onehot, axis=1, keepdims=True)
    gate_ref[...] = gate.reshape(S // LANES, LANES)
    counts = jnp.sum(onehot, axis=0, keepdims=True)
    me = jnp.sum(probs, axis=0, keepdims=True) / S
    laux_ref[...] = jnp.sum(me * (counts / S), axis=1, keepdims=True) * E


def _gating(x, wg_pad):
    return pl.pallas_call(
        _gating_body,
        out_shape=(
            jax.ShapeDtypeStruct((S // LANES, LANES), jnp.int32),
            jax.ShapeDtypeStruct((S // LANES, LANES), jnp.float32),
            jax.ShapeDtypeStruct((1, 1), jnp.float32),
        ),
    )(x, wg_pad)


def _dispatch_body(dst_hbm, gate_hbm, feat_hbm, disp_hbm, scale_hbm,
                   dst_v, gate_v, slot_v, scale_v, rows_v, sem):
    wid = lax.axis_index("s") * 2 + lax.axis_index("c")
    base = wid * ROWS_DISP
    pltpu.sync_copy(dst_hbm, dst_v)
    pltpu.sync_copy(gate_hbm, gate_v)
    for j in range(ROWS_DISP // L):
        dflt = (lax.iota(jnp.int32, L) + (base + j * L)) & (S - 1)
        slot_v[pl.ds(j * L, L)] = dflt

    def build(i, _):
        idx = dst_v[pl.ds(i * L, L)] - base
        m = (idx >= 0) & (idx < ROWS_DISP)
        tok = lax.iota(jnp.int32, L) + i * L
        plsc.store_scatter(slot_v, [idx], tok, mask=m)
        return 0
    lax.fori_loop(0, S // L, build, 0)

    for j in range(ROWS_DISP // L):
        sidx = slot_v[pl.ds(j * L, L)]
        scale_v[pl.ds(j * L, L)] = plsc.load_gather(gate_v, [sidx])
    pltpu.sync_copy(scale_v, scale_hbm.at[pl.ds(base, ROWS_DISP)])

    pltpu.async_copy(feat_hbm.at[slot_v], rows_v, sem).wait()
    pltpu.sync_copy(rows_v, disp_hbm.at[pl.ds(base, ROWS_DISP)])


def _dispatch(dst, gate, feat):
    mesh = plsc.VectorSubcoreMesh(core_axis_name="c", subcore_axis_name="s")
    return pl.kernel(
        _dispatch_body,
        out_type=(
            jax.ShapeDtypeStruct((EC, M), jnp.float32),
            jax.ShapeDtypeStruct((EC,), jnp.float32),
        ),
        mesh=mesh,
        compiler_params=pltpu.CompilerParams(needs_layout_passes=False),
        scratch_types=[
            pltpu.VMEM((S,), jnp.int32),
            pltpu.VMEM((S,), jnp.float32),
            pltpu.VMEM((ROWS_DISP,), jnp.int32),
            pltpu.VMEM((ROWS_DISP,), jnp.float32),
            pltpu.VMEM((ROWS_DISP, M), jnp.float32),
            pltpu.SemaphoreType.DMA,
        ],
    )(dst, gate, feat)


def _ffn_body(x_ref, w1_ref, b1_ref, w2_ref, b2_ref, sc_ref, o_ref, acc_ref):
    k = pl.program_id(1)

    @pl.when(k == 0)
    def _():
        acc_ref[...] = jnp.zeros_like(acc_ref)

    h = jnp.dot(x_ref[...].astype(jnp.bfloat16), w1_ref[0].astype(jnp.bfloat16),
                preferred_element_type=jnp.float32)
    h = jnp.maximum(h + b1_ref[0, 0, 0][None, :], 0.0)
    acc_ref[...] += jnp.dot(h.astype(jnp.bfloat16), w2_ref[0].astype(jnp.bfloat16),
                            preferred_element_type=jnp.float32)

    @pl.when(k == KM - 1)
    def _():
        o_ref[...] = (acc_ref[...] + b2_ref[0, 0][None, :]) * sc_ref[...]


def _ffn(disp, w1, b1, w2, b2, scale):
    ce = lambda e: jnp.minimum(e, E - 1)
    return pl.pallas_call(
        _ffn_body,
        grid=(E + 1, KM),
        in_specs=[
            pl.BlockSpec((C, M), lambda e, k: (ce(e), 0)),
            pl.BlockSpec((1, M, BM), lambda e, k: (ce(e), 0, k)),
            pl.BlockSpec((1, 1, 1, BM), lambda e, k: (ce(e), k, 0, 0)),
            pl.BlockSpec((1, BM, OUT), lambda e, k: (ce(e), k, 0)),
            pl.BlockSpec((1, 1, OUT), lambda e, k: (ce(e), 0, 0)),
            pl.BlockSpec((C, 1), lambda e, k: (e, 0)),
        ],
        out_specs=pl.BlockSpec((C, OUT), lambda e, k: (e, 0)),
        out_shape=jax.ShapeDtypeStruct((EC + C, OUT), jnp.float32),
        scratch_shapes=[pltpu.VMEM((C, OUT), jnp.float32)],
    )(disp, w1, b1.reshape(E, KM, 1, BM), w2, b2.reshape(E, 1, OUT), scale)


def _combine_body(dst_hbm, sout_hbm, out_hbm, idx_v, rows_v, sem):
    wid = lax.axis_index("s") * 2 + lax.axis_index("c")
    base = wid * ROWS_COMB
    pltpu.sync_copy(dst_hbm.at[pl.ds(base, ROWS_COMB)], idx_v)
    pltpu.async_copy(sout_hbm.at[idx_v], rows_v, sem).wait()
    pltpu.sync_copy(rows_v, out_hbm.at[pl.ds(base, ROWS_COMB)])


def _combine(dst, sout):
    mesh = plsc.VectorSubcoreMesh(core_axis_name="c", subcore_axis_name="s")
    return pl.kernel(
        _combine_body,
        out_type=jax.ShapeDtypeStruct((S, OUT), jnp.float32),
        mesh=mesh,
        compiler_params=pltpu.CompilerParams(needs_layout_passes=False),
        scratch_types=[
            pltpu.VMEM((ROWS_COMB,), jnp.int32),
            pltpu.VMEM((ROWS_COMB, OUT), jnp.float32),
            pltpu.SemaphoreType.DMA,
        ],
    )(dst, sout)


@jax.jit
def kernel(hidden_states, Wg, W1, b1, W2, b2):
    b, t, m = hidden_states.shape
    feat = hidden_states.reshape(S, M)
    wg_pad = jnp.zeros((M, LANES), jnp.float32).at[:, :E].set(Wg)
    dst2, gate2, laux = _gating(feat, wg_pad)
    dst = dst2.reshape(S)
    disp, scale = _dispatch(dst, gate2.reshape(S), feat)
    scale_full = jnp.concatenate([scale, jnp.zeros((C,), jnp.float32)])
    sout = _ffn(disp, W1, b1, W2, b2, scale_full.reshape(EC + C, 1))
    combined = _combine(dst, sout)
    return combined.reshape(b, t, OUT), laux.reshape(())
